# baseline (speedup 1.0000x reference)
import jax
import jax.numpy as jnp
from jax import lax
from jax.experimental import pallas as pl
from jax.experimental.pallas import tpu as pltpu
from jax.experimental.pallas import tpu_sc as plsc

VOCAB_SIZE = 1000000
EMB_SIZE = 32
N = 16384

NUM_CORES = 2
NUM_SUBCORES = 16
NUM_WORKERS = NUM_CORES * NUM_SUBCORES
LANES = 16


def _gather_body(table_hbm, vocab_hbm, out_hbm, idx_v, vals_v, sem):
    wid = lax.axis_index("s") * NUM_CORES + lax.axis_index("c")
    pltpu.sync_copy(vocab_hbm, idx_v)
    row_base = wid * VOCAB_SIZE

    def add_base(g, _):
        idx_v[pl.ds(g * LANES, LANES)] = (
            idx_v[pl.ds(g * LANES, LANES)] + row_base
        )
        return _

    lax.fori_loop(0, N // LANES, add_base, 0)
    pltpu.async_copy(table_hbm.at[idx_v], vals_v, sem).wait()
    pltpu.sync_copy(vals_v, out_hbm.at[pl.ds(wid * N, N)])


def _node_gather(table_flat_t, vocab_ids):
    mesh = plsc.VectorSubcoreMesh(
        core_axis_name="c", subcore_axis_name="s",
        num_cores=NUM_CORES, num_subcores=NUM_SUBCORES,
    )
    return pl.kernel(
        _gather_body,
        out_type=jax.ShapeDtypeStruct((EMB_SIZE * N,), jnp.float32),
        mesh=mesh,
        scratch_types=[
            pltpu.VMEM((N,), jnp.int32),
            pltpu.VMEM((N,), jnp.float32),
            pltpu.SemaphoreType.DMA,
        ],
    )(table_flat_t, vocab_ids)


@jax.jit
def _impl(vocab_ids, selector_ids, node_table, sel_table):
    vidx = vocab_ids.astype(jnp.int32)
    flat_t = node_table.T.reshape(EMB_SIZE * VOCAB_SIZE)
    out_t = _node_gather(flat_t, vidx)
    nodes = out_t.reshape(EMB_SIZE, N).T
    sel = jnp.take(sel_table, selector_ids.astype(jnp.int32), axis=0)
    return jnp.concatenate([nodes, sel], axis=1)


def kernel(vocab_ids, selector_ids, node_table, sel_table):
    return _impl(vocab_ids, selector_ids, node_table, sel_table)


# speedup vs baseline: 4.5567x; 4.5567x over previous
import jax
import jax.numpy as jnp
from jax import lax
from jax.experimental import pallas as pl
from jax.experimental.pallas import tpu as pltpu
from jax.experimental.pallas import tpu_sc as plsc

VOCAB_SIZE = 1000000
EMB_SIZE = 32
PAD_W = 128
N = 16384

NUM_CORES = 2
NUM_SUBCORES = 16
NUM_WORKERS = NUM_CORES * NUM_SUBCORES
ROWS_PER_WORKER = N // NUM_WORKERS


def _gather_body(table_hbm, vocab_hbm, out_hbm, idx_v, rows_v, sem):
    wid = lax.axis_index("s") * NUM_CORES + lax.axis_index("c")
    base = wid * ROWS_PER_WORKER
    pltpu.sync_copy(vocab_hbm.at[pl.ds(base, ROWS_PER_WORKER)], idx_v)
    pltpu.async_copy(table_hbm.at[idx_v], rows_v, sem).wait()
    pltpu.sync_copy(rows_v, out_hbm.at[pl.ds(base, ROWS_PER_WORKER)])


def _node_gather(table_pad, vocab_ids):
    mesh = plsc.VectorSubcoreMesh(
        core_axis_name="c", subcore_axis_name="s",
        num_cores=NUM_CORES, num_subcores=NUM_SUBCORES,
    )
    return pl.kernel(
        _gather_body,
        out_type=jax.ShapeDtypeStruct((N, PAD_W), jnp.float32),
        mesh=mesh,
        scratch_types=[
            pltpu.VMEM((ROWS_PER_WORKER,), jnp.int32),
            pltpu.VMEM((ROWS_PER_WORKER, PAD_W), jnp.float32),
            pltpu.SemaphoreType.DMA,
        ],
    )(table_pad, vocab_ids)


@jax.jit
def _impl(vocab_ids, selector_ids, node_table, sel_table):
    vidx = vocab_ids.astype(jnp.int32)
    table_pad = jnp.pad(node_table, ((0, 0), (0, PAD_W - EMB_SIZE)))
    out = _node_gather(table_pad, vidx)
    sel = jnp.take(sel_table, selector_ids.astype(jnp.int32), axis=0)
    return jnp.concatenate([out[:, :EMB_SIZE], sel], axis=1)


def kernel(vocab_ids, selector_ids, node_table, sel_table):
    return _impl(vocab_ids, selector_ids, node_table, sel_table)
